# Initial kernel scaffold; baseline (speedup 1.0000x reference)
import jax
import jax.numpy as jnp
from jax.experimental import pallas as pl


def kernel(layer0_code, layer0_embed, last_talker_hidden, lm_head_weights):
    raise NotImplementedError("write your pallas kernel here")


# speedup vs baseline: 5.0829x; 5.0829x over previous
import numpy as np
import jax
import jax.numpy as jnp
from jax.experimental import pallas as pl
from jax.experimental.pallas import tpu as pltpu

TOPK = 50
INV_T = np.float32(1.0 / max(0.9, 1e-06))
TINY = np.float32(np.finfo(np.float32).tiny)
ONE_MINUS_TINY = np.float32(np.float64(1.0) - np.float64(TINY))
IMIN = np.int32(-2147483648)

_BITS = [np.int32((1 << b) - ((1 << 32) if b == 31 else 0)) for b in range(32)]


def _mm_body(h_ref, w_ref, o_ref):
    acc = jax.lax.dot_general(
        h_ref[...], w_ref[0],
        dimension_numbers=(((1,), (1,)), ((), ())),
        preferred_element_type=jnp.float32)
    o_ref[0] = acc * INV_T


def _threefry_gumbel(k0, k1, p):
    ks2 = k0 ^ k1 ^ np.int32(0x1BD11BDA)
    ks = [k0, k1, ks2]
    rot = ((13, 15, 26, 6), (17, 29, 16, 24))
    x0 = jnp.full_like(p, k0)
    x1 = p + k1
    for i in range(5):
        for r in rot[i % 2]:
            x0 = x0 + x1
            x1 = (jax.lax.shift_left(x1, np.int32(r))
                  | jax.lax.shift_right_logical(x1, np.int32(32 - r)))
            x1 = x1 ^ x0
        x0 = x0 + ks[(i + 1) % 3]
        x1 = x1 + ks[(i + 2) % 3] + np.int32(i + 1)
    bits = x0 ^ x1
    fb = jax.lax.shift_right_logical(bits, np.int32(9)) | np.int32(0x3F800000)
    u0 = jax.lax.bitcast_convert_type(fb, jnp.float32) - np.float32(1.0)
    u = jnp.maximum(TINY, u0 * ONE_MINUS_TINY + TINY)
    return -jnp.log(-jnp.log(u))


def _sample_body(x_ref, sk_ref, o_ref):
    s = pl.program_id(0)
    rb = pl.program_id(1)
    x = x_ref[0]
    rows, v = x.shape

    b = jax.lax.bitcast_convert_type(x, jnp.int32)
    t = jnp.where(b >= 0, b, b ^ np.int32(0x7FFFFFFF))

    res_u = jnp.zeros((rows, 1), jnp.int32)
    for bit in range(31, -1, -1):
        cand_u = res_u | _BITS[bit]
        cand_s = cand_u ^ IMIN
        cnt = jnp.sum((t >= cand_s).astype(jnp.int32), axis=1, keepdims=True)
        res_u = jnp.where(cnt >= TOPK, cand_u, res_u)
    thresh_s = res_u ^ IMIN
    mask = t >= thresh_s

    col = jax.lax.broadcasted_iota(jnp.int32, (rows, v), 1)
    row = jax.lax.broadcasted_iota(jnp.int32, (rows, v), 0) + rb * rows
    p = row * np.int32(v) + col
    g = _threefry_gumbel(sk_ref[s, 0], sk_ref[s, 1], p)

    total = jnp.where(mask, x, -jnp.inf) + g
    m = jnp.max(total, axis=1, keepdims=True)
    win = jnp.min(jnp.where(total == m, col, np.int32(v)), axis=1)
    o_ref[0, 0] = win


def kernel(layer0_code, layer0_embed, last_talker_hidden, lm_head_weights):
    hidden = last_talker_hidden
    bsz, h = hidden.shape
    steps, vocab, _ = lm_head_weights.shape

    key = jax.random.key(42)
    sks = []
    for _ in range(steps):
        key, sk = jax.random.split(key)
        sks.append(jax.random.key_data(sk))
    skd = jax.lax.bitcast_convert_type(jnp.stack(sks), jnp.int32)

    r_mm, vc = 512, 512
    scaled = pl.pallas_call(
        _mm_body,
        grid=(steps, bsz // r_mm, vocab // vc),
        in_specs=[
            pl.BlockSpec((r_mm, h), lambda s, i, j: (i, 0)),
            pl.BlockSpec((1, vc, h), lambda s, i, j: (s, j, 0)),
        ],
        out_specs=pl.BlockSpec((1, r_mm, vc), lambda s, i, j: (s, i, j)),
        out_shape=jax.ShapeDtypeStruct((steps, bsz, vocab), jnp.float32),
        compiler_params=pltpu.CompilerParams(
            dimension_semantics=("parallel", "parallel", "parallel")),
    )(hidden, lm_head_weights)

    r_ep = 128
    nrb = bsz // r_ep
    codes = pl.pallas_call(
        _sample_body,
        grid=(steps, nrb),
        in_specs=[
            pl.BlockSpec((1, r_ep, vocab), lambda s, i: (s, i, 0)),
            pl.BlockSpec((steps, 2), lambda s, i: (0, 0),
                         memory_space=pltpu.SMEM),
        ],
        out_specs=pl.BlockSpec((1, 1, r_ep), lambda s, i: (s * nrb + i, 0, 0)),
        out_shape=jax.ShapeDtypeStruct((steps * nrb, 1, r_ep), jnp.int32),
        compiler_params=pltpu.CompilerParams(
            dimension_semantics=("arbitrary", "arbitrary")),
    )(scaled, skd)

    codes = codes.reshape(steps, bsz).T
    return jnp.concatenate(
        [layer0_code.reshape(bsz, 1).astype(jnp.int32), codes], axis=1)
